# baseline (speedup 1.0000x reference)
import functools

import jax
import jax.numpy as jnp
from jax import lax
from jax.experimental import pallas as pl
from jax.experimental.pallas import tpu as pltpu


def _scan_body(bias_ref, hid_ref, w_ref, scores_ref):
    part = lax.dot_general(
        w_ref[...], hid_ref[...],
        dimension_numbers=(((1,), (1,)), ((), ())),
        preferred_element_type=jnp.float32,
    )
    scores_ref[0] = part + bias_ref[0]


def _select_body(k_ref, scores_ref, mask_ref):
    s = scores_ref[...]
    B = s.shape[0]
    u = lax.bitcast_convert_type(s, jnp.uint32)
    neg = u >= jnp.uint32(0x80000000)
    key = jnp.where(neg, ~u, u | jnp.uint32(0x80000000))
    kk = k_ref[...]

    def bit_step(i, th):
        cand = th | (jnp.uint32(1) << (31 - i).astype(jnp.uint32))
        cnt = jnp.sum((key >= cand).astype(jnp.int32), axis=1, keepdims=True)
        return jnp.where(cnt >= kk, cand, th)

    th = lax.fori_loop(0, 32, bit_step, jnp.zeros((B, 1), jnp.uint32),
                       unroll=True)

    gt = key > th
    tie = key == th
    need = kk - jnp.sum(gt.astype(jnp.int32), axis=1, keepdims=True)
    idxs = lax.broadcasted_iota(jnp.int32, s.shape, 1)

    def idx_step(i, r):
        cand = r + (jnp.int32(1) << (11 - i))
        cnt = jnp.sum((tie & (idxs < cand)).astype(jnp.int32),
                      axis=1, keepdims=True)
        return jnp.where(cnt < need, cand, r)

    r = lax.fori_loop(0, 12, idx_step, jnp.zeros((B, 1), jnp.int32),
                      unroll=True)

    mask_ref[...] = (gt | (tie & (idxs <= r))).astype(jnp.int8)


def kernel(hidden, keep_ratio, W, b):
    B, T, C = hidden.shape
    T_BLK = 256
    N = (B * T) // T_BLK

    kr = jnp.clip(keep_ratio, 0.1, 1.0)
    k = jnp.maximum(1, (kr * T).astype(jnp.int32))
    w_row = W.reshape(1, C)
    hid2d = hidden.reshape(B * T, C)

    scores3 = pl.pallas_call(
        _scan_body,
        grid=(N,),
        in_specs=[
            pl.BlockSpec(memory_space=pltpu.SMEM),
            pl.BlockSpec((T_BLK, C), lambda i: (i, 0)),
            pl.BlockSpec((1, C), lambda i: (0, 0)),
        ],
        out_specs=pl.BlockSpec((1, 1, T_BLK), lambda i: (i, 0, 0)),
        out_shape=jax.ShapeDtypeStruct((N, 1, T_BLK), jnp.float32),
        compiler_params=pltpu.CompilerParams(
            dimension_semantics=("arbitrary",),
        ),
    )(b, hid2d, w_row)
    scores = scores3.reshape(B, T)

    mask_i8 = pl.pallas_call(
        _select_body,
        in_specs=[
            pl.BlockSpec((B, 1), lambda: (0, 0)),
            pl.BlockSpec((B, T), lambda: (0, 0)),
        ],
        out_specs=pl.BlockSpec((B, T), lambda: (0, 0)),
        out_shape=jax.ShapeDtypeStruct((B, T), jnp.int8),
    )(k.reshape(B, 1), scores)

    return (mask_i8.astype(jnp.bool_), scores)


# speedup vs baseline: 1.0797x; 1.0797x over previous
import functools

import jax
import jax.numpy as jnp
from jax import lax
from jax.experimental import pallas as pl
from jax.experimental.pallas import tpu as pltpu


def _fused_body(k_ref, bias_ref, hid_ref, w_ref, scores_ref, mask_ref,
                *, t_blk, n_steps, chunks_per_row, n_rows, t_total):
    i = pl.program_id(0)

    part = lax.dot_general(
        w_ref[...], hid_ref[...],
        dimension_numbers=(((1,), (1,)), ((), ())),
        preferred_element_type=jnp.float32,
    )
    scores_ref[i] = part + bias_ref[0]

    idx_bits = (t_total - 1).bit_length()
    if idx_bits % 2:
        idx_bits += 1

    def select_row(r):
        s = scores_ref[pl.ds(r * chunks_per_row, chunks_per_row), 0, :]
        u = lax.bitcast_convert_type(s, jnp.uint32)
        neg = u >= jnp.uint32(0x80000000)
        key = jnp.where(neg, ~u, u | jnp.uint32(0x80000000))
        kk = k_ref[r]

        th = jnp.uint32(0)
        for shift in range(30, -2, -2):
            d = jnp.int32(0)
            for c in (1, 2, 3):
                cnt = jnp.sum((key >= (th | jnp.uint32(c << shift)))
                              .astype(jnp.int32))
                d = d + (cnt >= kk).astype(jnp.int32)
            th = th | (d.astype(jnp.uint32) << shift)

        gt = key > th
        tie = key == th
        need = kk - jnp.sum(gt.astype(jnp.int32))
        gidx = (lax.broadcasted_iota(jnp.int32, s.shape, 0) * t_blk
                + lax.broadcasted_iota(jnp.int32, s.shape, 1))

        rsel = jnp.int32(0)
        for shift in range(idx_bits - 2, -2, -2):
            d = jnp.int32(0)
            for c in (1, 2, 3):
                cnt = jnp.sum((tie & (gidx < (rsel + jnp.int32(c << shift))))
                              .astype(jnp.int32))
                d = d + (cnt < need).astype(jnp.int32)
            rsel = rsel + (d << shift)

        sel = gt | (tie & (gidx <= rsel))
        mask_ref[pl.ds(r * chunks_per_row, chunks_per_row), 0, :] = (
            sel.astype(jnp.int32))

    for r in range(n_rows):
        trigger = min((r + 1) * chunks_per_row, n_steps - 1)

        @pl.when(i == trigger)
        def _(r=r):
            select_row(r)


def kernel(hidden, keep_ratio, W, b):
    B, T, C = hidden.shape
    T_BLK = 512
    N = (B * T) // T_BLK
    CPR = T // T_BLK

    kr = jnp.clip(keep_ratio, 0.1, 1.0)
    k = jnp.maximum(1, (kr * T).astype(jnp.int32))
    w_row = W.reshape(1, C)
    hid2d = hidden.reshape(B * T, C)

    scores3, mask3 = pl.pallas_call(
        functools.partial(_fused_body, t_blk=T_BLK, n_steps=N,
                          chunks_per_row=CPR, n_rows=B, t_total=T),
        grid=(N,),
        in_specs=[
            pl.BlockSpec(memory_space=pltpu.SMEM),
            pl.BlockSpec(memory_space=pltpu.SMEM),
            pl.BlockSpec((T_BLK, C), lambda i: (i, 0)),
            pl.BlockSpec((1, C), lambda i: (0, 0)),
        ],
        out_specs=[
            pl.BlockSpec((N, 1, T_BLK), lambda i: (0, 0, 0)),
            pl.BlockSpec((N, 1, T_BLK), lambda i: (0, 0, 0)),
        ],
        out_shape=[
            jax.ShapeDtypeStruct((N, 1, T_BLK), jnp.float32),
            jax.ShapeDtypeStruct((N, 1, T_BLK), jnp.int32),
        ],
        compiler_params=pltpu.CompilerParams(
            dimension_semantics=("arbitrary",),
        ),
    )(k, b, hid2d, w_row)

    return (mask3.reshape(B, T).astype(jnp.bool_), scores3.reshape(B, T))


# speedup vs baseline: 1.2030x; 1.1142x over previous
import functools

import jax
import jax.numpy as jnp
from jax import lax
from jax.experimental import pallas as pl
from jax.experimental.pallas import tpu as pltpu


def _scan_body(bias_ref, hid_ref, w_ref, scores_ref):
    part = lax.dot_general(
        w_ref[...], hid_ref[...],
        dimension_numbers=(((1,), (1,)), ((), ())),
        preferred_element_type=jnp.float32,
    )
    scores_ref[0] = part + bias_ref[0]


def _select_body(k_ref, scores_ref, mask_ref, *, idx_bits):
    s = scores_ref[...]
    B = s.shape[0]
    u = lax.bitcast_convert_type(s, jnp.uint32)
    neg = u >= jnp.uint32(0x80000000)
    key = jnp.where(neg, ~u, u | jnp.uint32(0x80000000))
    kk = k_ref[...]

    th = jnp.zeros((B, 1), jnp.uint32)
    for shift in range(30, -2, -2):
        d = jnp.zeros((B, 1), jnp.int32)
        for c in (1, 2, 3):
            cnt = jnp.sum((key >= (th | jnp.uint32(c << shift)))
                          .astype(jnp.int32), axis=1, keepdims=True)
            d = d + (cnt >= kk).astype(jnp.int32)
        th = th | (d.astype(jnp.uint32) << shift)

    gt = key > th
    tie = key == th
    need = kk - jnp.sum(gt.astype(jnp.int32), axis=1, keepdims=True)
    idxs = lax.broadcasted_iota(jnp.int32, s.shape, 1)

    rsel = jnp.zeros((B, 1), jnp.int32)
    for shift in range(idx_bits - 2, -2, -2):
        d = jnp.zeros((B, 1), jnp.int32)
        for c in (1, 2, 3):
            cnt = jnp.sum((tie & (idxs < (rsel + jnp.int32(c << shift))))
                          .astype(jnp.int32), axis=1, keepdims=True)
            d = d + (cnt < need).astype(jnp.int32)
        rsel = rsel + (d << shift)

    mask_ref[...] = (gt | (tie & (idxs <= rsel))).astype(jnp.int32)


def kernel(hidden, keep_ratio, W, b):
    B, T, C = hidden.shape
    T_BLK = 512
    N = (B * T) // T_BLK
    idx_bits = (T - 1).bit_length()
    if idx_bits % 2:
        idx_bits += 1

    kr = jnp.clip(keep_ratio, 0.1, 1.0)
    k = jnp.maximum(1, (kr * T).astype(jnp.int32))
    w_row = W.reshape(1, C)
    hid2d = hidden.reshape(B * T, C)

    scores3 = pl.pallas_call(
        _scan_body,
        grid=(N,),
        in_specs=[
            pl.BlockSpec(memory_space=pltpu.SMEM),
            pl.BlockSpec((T_BLK, C), lambda i: (i, 0)),
            pl.BlockSpec((1, C), lambda i: (0, 0)),
        ],
        out_specs=pl.BlockSpec((1, 1, T_BLK), lambda i: (i, 0, 0)),
        out_shape=jax.ShapeDtypeStruct((N, 1, T_BLK), jnp.float32),
        compiler_params=pltpu.CompilerParams(
            dimension_semantics=("arbitrary",),
        ),
    )(b, hid2d, w_row)
    scores = scores3.reshape(B, T)

    mask_i32 = pl.pallas_call(
        functools.partial(_select_body, idx_bits=idx_bits),
        in_specs=[
            pl.BlockSpec((B, 1), lambda: (0, 0)),
            pl.BlockSpec((B, T), lambda: (0, 0)),
        ],
        out_specs=pl.BlockSpec((B, T), lambda: (0, 0)),
        out_shape=jax.ShapeDtypeStruct((B, T), jnp.int32),
    )(k.reshape(B, 1), scores)

    return (mask_i32.astype(jnp.bool_), scores)
